# Initial kernel scaffold; baseline (speedup 1.0000x reference)
import jax
import jax.numpy as jnp
from jax.experimental import pallas as pl


def kernel(x, timestamps, pe, hourly_table, daily_table):
    raise NotImplementedError("write your pallas kernel here")


# speedup vs baseline: 4.2342x; 4.2342x over previous
import jax
import jax.numpy as jnp
from jax.experimental import pallas as pl

_S_BLK = 512


def _pe_add_kernel(x_ref, pe_ref, h_ref, d_ref, o_ref):
    bias = h_ref[0, :] + d_ref[0, :]
    o_ref[0] = x_ref[0] + (pe_ref[0] + bias[None, :])


def kernel(x, timestamps, pe, hourly_table, daily_table):
    B, S, D = x.shape
    d_blk = D // 4
    s_blk = _S_BLK if S % _S_BLK == 0 else S
    grid = (S // s_blk, 4, B)
    return pl.pallas_call(
        _pe_add_kernel,
        grid=grid,
        in_specs=[
            pl.BlockSpec((1, s_blk, d_blk), lambda i, j, b: (b, i, j)),
            pl.BlockSpec((1, s_blk, d_blk), lambda i, j, b: (0, i, j)),
            pl.BlockSpec(hourly_table.shape, lambda i, j, b: (0, 0)),
            pl.BlockSpec(daily_table.shape, lambda i, j, b: (0, 0)),
        ],
        out_specs=pl.BlockSpec((1, s_blk, d_blk), lambda i, j, b: (b, i, j)),
        out_shape=jax.ShapeDtypeStruct((B, S, D), x.dtype),
    )(x, pe, hourly_table, daily_table)


# speedup vs baseline: 7.6536x; 1.8076x over previous
import jax
import jax.numpy as jnp
from jax.experimental import pallas as pl

_S_BLK = 512


def _pe_add_kernel(x_ref, pe_ref, h_ref, d_ref, o_ref):
    bias = h_ref[0, :] + d_ref[0, :]
    w = bias.shape[0]
    for k in range(4):
        sl = slice(k * w, (k + 1) * w)
        o_ref[0, :, sl] = x_ref[0, :, sl] + (pe_ref[0, :, sl] + bias[None, :])


def kernel(x, timestamps, pe, hourly_table, daily_table):
    B, S, D = x.shape
    s_blk = _S_BLK if S % _S_BLK == 0 else S
    grid = (S // s_blk, B)
    return pl.pallas_call(
        _pe_add_kernel,
        grid=grid,
        in_specs=[
            pl.BlockSpec((1, s_blk, D), lambda i, b: (b, i, 0)),
            pl.BlockSpec((1, s_blk, D), lambda i, b: (0, i, 0)),
            pl.BlockSpec(hourly_table.shape, lambda i, b: (0, 0)),
            pl.BlockSpec(daily_table.shape, lambda i, b: (0, 0)),
        ],
        out_specs=pl.BlockSpec((1, s_blk, D), lambda i, b: (b, i, 0)),
        out_shape=jax.ShapeDtypeStruct((B, S, D), x.dtype),
    )(x, pe, hourly_table, daily_table)
